# baseline (speedup 1.0000x reference)
import functools

import jax
import jax.numpy as jnp
from jax import lax
from jax.experimental import pallas as pl
from jax.experimental.pallas import tpu as pltpu
from jax.experimental.pallas import tpu_sc as plsc

B, C, H, W = 1, 512, 64, 64
HW = H * W
C2 = C // 2
RT = 512
NT = HW // RT
NEG_INF = float("-inf")
LANES = 16
NCH = HW // LANES


NB = HW // 128
MASK_BIAS = -1e30


def _prescale_body(lw_ref, lwf_ref, fl_ref, flf_ref,
                   s1_ref, b1_ref, s2_ref, b2_ref):
    def side(lw, flags, s_ref, b_ref):
        nl = jnp.sum(lw * lw, axis=0)
        s_ref[...] = (1.0 / jnp.sqrt(nl))[None, :]
        b_ref[...] = jnp.where(flags == 0, 0.0, MASK_BIAS)

    side(lw_ref[...], fl_ref[...], s1_ref, b1_ref)
    side(lwf_ref[...], flf_ref[...], s2_ref, b2_ref)


def _stage0(lwT, lwfT, fl2d, flf2d):
    outs = [jax.ShapeDtypeStruct((1, HW), jnp.float32)] * 4
    return pl.pallas_call(_prescale_body, out_shape=outs)(
        lwT, lwfT, fl2d, flf2d)


def _cos_argmax_body(fw_ref, lw_ref, lwf_ref, s1_ref, b1_ref, s2_ref, b2_ref,
                     rm_ref, bq_ref, rmf_ref, bqf_ref):
    fw_t = fw_ref[...]
    nf = jnp.sum(fw_t * fw_t, axis=0)
    rnf = 1.0 / jnp.sqrt(nf)
    lane = lax.broadcasted_iota(jnp.int32, (RT, 128), 1)

    def one_side(lw, scale, bias):
        num = lax.dot_general(fw_t, lw, (((0,), (0,)), ((), ())),
                              preferred_element_type=jnp.float32)
        m_acc = jnp.full((RT, 128), NEG_INF, jnp.float32)
        b_acc = jnp.zeros((RT, 128), jnp.int32)
        for b in range(NB):
            sl = slice(b * 128, (b + 1) * 128)
            x = num[:, sl] * scale[0:1, sl] + bias[0:1, sl]
            better = x > m_acc
            m_acc = jnp.where(better, x, m_acc)
            b_acc = jnp.where(better, b, b_acc)
        i_acc = b_acc * 128 + lane
        m_t = m_acc.T
        i_t = i_acc.T
        m = jnp.max(m_t, axis=0)
        idx = jnp.min(jnp.where(m_t == m[None, :], i_t, HW), axis=0)
        return m * rnf, idx

    m, idx = one_side(lw_ref[...], s1_ref[...], b1_ref[...])
    mf, idxf = one_side(lwf_ref[...], s2_ref[...], b2_ref[...])
    rm_ref[...] = m.reshape(1, 1, RT)
    bq_ref[...] = idx.reshape(1, 1, RT)
    rmf_ref[...] = mf.reshape(1, 1, RT)
    bqf_ref[...] = idxf.reshape(1, 1, RT)


def _stage1(fwT, lwT, lwfT, s1, b1, s2, b2):
    f32 = jnp.float32
    outs = [jax.ShapeDtypeStruct((NT, 1, RT), f32),
            jax.ShapeDtypeStruct((NT, 1, RT), jnp.int32),
            jax.ShapeDtypeStruct((NT, 1, RT), f32),
            jax.ShapeDtypeStruct((NT, 1, RT), jnp.int32)]
    in_specs = [
        pl.BlockSpec((C2, RT), lambda i: (0, i)),
        pl.BlockSpec((C2, HW), lambda i: (0, 0)),
        pl.BlockSpec((C2, HW), lambda i: (0, 0)),
        pl.BlockSpec((1, HW), lambda i: (0, 0)),
        pl.BlockSpec((1, HW), lambda i: (0, 0)),
        pl.BlockSpec((1, HW), lambda i: (0, 0)),
        pl.BlockSpec((1, HW), lambda i: (0, 0)),
    ]
    out_specs = [pl.BlockSpec((1, 1, RT), lambda i: (i, 0, 0))] * 4
    return pl.pallas_call(
        _cos_argmax_body, grid=(NT,), in_specs=in_specs,
        out_specs=out_specs, out_shape=outs,
    )(fwT, lwT, lwfT, s1, b1, s2, b2)


def _stage2_body(rm_hbm, rmf_hbm, bq_hbm, bqf_hbm, fl_hbm, flf_hbm,
                 orm_hbm, ormf_hbm, ocm_hbm, ocmf_hbm,
                 rm_v, rmf_v, bq_v, bqf_v, fl_v, flf_v,
                 rank1_v, compf_v, o_rm, o_rmf, o_cm, o_cmf):
    cid = lax.axis_index("c")
    sid = lax.axis_index("s")

    @pl.when(jnp.logical_and(cid == 0, sid == 0))
    def _():
        pltpu.sync_copy(rm_hbm, rm_v)
        pltpu.sync_copy(rmf_hbm, rmf_v)
        pltpu.sync_copy(bq_hbm, bq_v)
        pltpu.sync_copy(bqf_hbm, bqf_v)
        pltpu.sync_copy(fl_hbm, fl_v)
        pltpu.sync_copy(flf_hbm, flf_v)

        zf = jnp.zeros((LANES,), jnp.float32)

        def zero_body(j, _):
            ds = pl.ds(j * LANES, LANES)
            o_cm[ds] = zf
            o_cmf[ds] = zf
            return 0

        lax.fori_loop(0, NCH, zero_body, 0, unroll=8)

        ones16 = jnp.ones((LANES,), jnp.float32)

        def p1(j, carry):
            c1, c2 = carry
            ds = pl.ds(j * LANES, LANES)
            flv = fl_v[ds]
            flfv = flf_v[ds]
            rank1 = plsc.cumsum(flv) - flv + c1
            rankf = plsc.cumsum(flfv) - flfv + c2
            rank1_v[ds] = rank1
            plsc.store_scatter(compf_v, [rankf], rmf_v[ds], mask=flfv == 1)
            plsc.store_scatter(o_cm, [bq_v[ds]], ones16, mask=flv == 1)
            plsc.store_scatter(o_cmf, [bqf_v[ds]], ones16, mask=flfv == 1)
            return (c1 + jnp.sum(flv), c2 + jnp.sum(flfv))

        lax.fori_loop(0, NCH, p1, (jnp.int32(0), jnp.int32(0)), unroll=4)

        def p2(j, _):
            ds = pl.ds(j * LANES, LANES)
            r1 = jnp.minimum(rank1_v[ds], HW - 1)
            cfv = plsc.load_gather(compf_v, [r1])
            olp = rm_v[ds] >= cfv
            flv = fl_v[ds]
            pix = lax.iota(jnp.int32, LANES) + j * LANES
            ok = jnp.logical_and(flv == 1, pix != 0)
            o_rm[ds] = jnp.where(jnp.logical_and(ok, olp), 1.0, 0.0)
            o_rmf[ds] = jnp.where(
                jnp.logical_and(ok, jnp.logical_not(olp)), 1.0, 0.0)
            return 0

        lax.fori_loop(0, NCH, p2, 0, unroll=4)

        pltpu.sync_copy(o_rm, orm_hbm)
        pltpu.sync_copy(o_rmf, ormf_hbm)
        pltpu.sync_copy(o_cm, ocm_hbm)
        pltpu.sync_copy(o_cmf, ocmf_hbm)


def _stage2(rm, rmf, bq, bqf, fl, flf):
    f32 = jnp.float32
    i32 = jnp.int32
    mesh = plsc.VectorSubcoreMesh(core_axis_name="c", subcore_axis_name="s")
    run = functools.partial(
        pl.kernel, mesh=mesh,
        compiler_params=pltpu.CompilerParams(needs_layout_passes=False),
        out_type=[jax.ShapeDtypeStruct((HW,), f32)] * 4,
        scratch_types=[
            pltpu.VMEM((HW,), f32),
            pltpu.VMEM((HW,), f32),
            pltpu.VMEM((HW,), i32),
            pltpu.VMEM((HW,), i32),
            pltpu.VMEM((HW,), i32),
            pltpu.VMEM((HW,), i32),
            pltpu.VMEM((HW,), i32),
            pltpu.VMEM((HW,), f32),
            pltpu.VMEM((HW,), f32),
            pltpu.VMEM((HW,), f32),
            pltpu.VMEM((HW,), f32),
            pltpu.VMEM((HW,), f32),
        ],
    )(_stage2_body)
    return run(rm, rmf, bq, bqf, fl, flf)


def _paste_body(x_ref, lwf_ref, rm_ref, rmf_ref, cm_ref, cmf_ref, out_ref):
    lw = x_ref[C2:, :]
    s = lax.dot_general(lw, cm_ref[...], (((1,), (1,)), ((), ())),
                        preferred_element_type=jnp.float32)
    sf = lax.dot_general(lwf_ref[...], cmf_ref[...], (((1,), (1,)), ((), ())),
                         preferred_element_type=jnp.float32)
    out_ref[:C, :] = x_ref[...]
    out_ref[C:, :] = s * rm_ref[...] + sf * rmf_ref[...]


def _stage3(xf, lwfT, rm2d, rmf2d, cm2d, cmf2d):
    return pl.pallas_call(
        _paste_body,
        out_shape=jax.ShapeDtypeStruct((C + C2, HW), jnp.float32),
    )(xf, lwfT, rm2d, rmf2d, cm2d, cmf2d)


def kernel(input, flip_feat, flag, flag_flip):
    xf = input[0].reshape(C, HW)
    fwT = xf[:C2]
    lwT = xf[C2:]
    lwfT = flip_feat[0].reshape(C2, HW)
    fl = flag[0].astype(jnp.int32)
    flf = flag_flip[0].astype(jnp.int32)

    s1, b1, s2, b2 = _stage0(lwT, lwfT, fl.reshape(1, HW), flf.reshape(1, HW))
    rm3, bq3, rmf3, bqf3 = _stage1(fwT, lwT, lwfT, s1, b1, s2, b2)
    rm = rm3.reshape(HW)
    bq = bq3.reshape(HW)
    rmf = rmf3.reshape(HW)
    bqf = bqf3.reshape(HW)

    orm, ormf, ocm, ocmf = _stage2(rm, rmf, bq, bqf, fl, flf)

    out = _stage3(xf, lwfT, orm.reshape(1, HW), ormf.reshape(1, HW),
                  ocm.reshape(1, HW), ocmf.reshape(1, HW))

    return out.reshape(1, C + C2, H, W)


# speedup vs baseline: 1.0573x; 1.0573x over previous
import functools

import jax
import jax.numpy as jnp
from jax import lax
from jax.experimental import pallas as pl
from jax.experimental.pallas import tpu as pltpu
from jax.experimental.pallas import tpu_sc as plsc

B, C, H, W = 1, 512, 64, 64
HW = H * W
C2 = C // 2
RT = 512
NT = HW // RT
NEG_INF = float("-inf")
LANES = 16
NCH = HW // LANES


NB = HW // 128
MASK_BIAS = -1e30


def _prescale_body(lw_ref, lwf_ref, fl_ref, flf_ref,
                   s1_ref, b1_ref, s2_ref, b2_ref):
    def side(lw, flags, s_ref, b_ref):
        nl = jnp.sum(lw * lw, axis=0)
        s_ref[...] = (1.0 / jnp.sqrt(nl))[None, :]
        b_ref[...] = jnp.where(flags == 0, 0.0, MASK_BIAS)

    side(lw_ref[...], fl_ref[...], s1_ref, b1_ref)
    side(lwf_ref[...], flf_ref[...], s2_ref, b2_ref)


def _stage0(lwT, lwfT, fl2d, flf2d):
    outs = [jax.ShapeDtypeStruct((1, HW), jnp.float32)] * 4
    return pl.pallas_call(_prescale_body, out_shape=outs)(
        lwT, lwfT, fl2d, flf2d)


def _cos_argmax_body(fw_ref, lw_ref, lwf_ref, s1_ref, b1_ref, s2_ref, b2_ref,
                     rm_ref, bq_ref, rmf_ref, bqf_ref):
    fw_t = fw_ref[...]
    nf = jnp.sum(fw_t * fw_t, axis=0)
    rnf = 1.0 / jnp.sqrt(nf)
    lane = lax.broadcasted_iota(jnp.int32, (RT, 128), 1)

    def one_side(lw, scale, bias):
        num = lax.dot_general(fw_t, lw, (((0,), (0,)), ((), ())),
                              preferred_element_type=jnp.float32)
        m_acc = jnp.full((RT, 128), NEG_INF, jnp.float32)
        b_acc = jnp.zeros((RT, 128), jnp.int32)
        for b in range(NB):
            sl = slice(b * 128, (b + 1) * 128)
            x = num[:, sl] * scale[0:1, sl] + bias[0:1, sl]
            better = x > m_acc
            m_acc = jnp.where(better, x, m_acc)
            b_acc = jnp.where(better, b, b_acc)
        i_acc = b_acc * 128 + lane
        m_t = m_acc.T
        i_t = i_acc.T
        m = jnp.max(m_t, axis=0)
        idx = jnp.min(jnp.where(m_t == m[None, :], i_t, HW), axis=0)
        return m * rnf, idx

    m, idx = one_side(lw_ref[...], s1_ref[...], b1_ref[...])
    mf, idxf = one_side(lwf_ref[...], s2_ref[...], b2_ref[...])
    rm_ref[...] = m.reshape(1, 1, RT)
    bq_ref[...] = idx.reshape(1, 1, RT)
    rmf_ref[...] = mf.reshape(1, 1, RT)
    bqf_ref[...] = idxf.reshape(1, 1, RT)


def _stage1(fwT, lwT, lwfT, s1, b1, s2, b2):
    f32 = jnp.float32
    outs = [jax.ShapeDtypeStruct((NT, 1, RT), f32),
            jax.ShapeDtypeStruct((NT, 1, RT), jnp.int32),
            jax.ShapeDtypeStruct((NT, 1, RT), f32),
            jax.ShapeDtypeStruct((NT, 1, RT), jnp.int32)]
    in_specs = [
        pl.BlockSpec((C2, RT), lambda i: (0, i)),
        pl.BlockSpec((C2, HW), lambda i: (0, 0)),
        pl.BlockSpec((C2, HW), lambda i: (0, 0)),
        pl.BlockSpec((1, HW), lambda i: (0, 0)),
        pl.BlockSpec((1, HW), lambda i: (0, 0)),
        pl.BlockSpec((1, HW), lambda i: (0, 0)),
        pl.BlockSpec((1, HW), lambda i: (0, 0)),
    ]
    out_specs = [pl.BlockSpec((1, 1, RT), lambda i: (i, 0, 0))] * 4
    return pl.pallas_call(
        _cos_argmax_body, grid=(NT,), in_specs=in_specs,
        out_specs=out_specs, out_shape=outs,
    )(fwT, lwT, lwfT, s1, b1, s2, b2)


def _stage2_body(rm_hbm, rmf_hbm, bq_hbm, bqf_hbm, fl_hbm, flf_hbm,
                 orm_hbm, ormf_hbm, ocm_hbm, ocmf_hbm,
                 rm_v, rmf_v, bq_v, bqf_v, fl_v, flf_v,
                 rank1_v, compf_v, o_rm, o_rmf, o_cm, o_cmf):
    cid = lax.axis_index("c")
    sid = lax.axis_index("s")

    @pl.when(jnp.logical_and(cid == 0, sid == 0))
    def _():
        pltpu.sync_copy(rm_hbm, rm_v)
        pltpu.sync_copy(rmf_hbm, rmf_v)
        pltpu.sync_copy(bq_hbm, bq_v)
        pltpu.sync_copy(bqf_hbm, bqf_v)
        pltpu.sync_copy(fl_hbm, fl_v)
        pltpu.sync_copy(flf_hbm, flf_v)

        zf = jnp.zeros((LANES,), jnp.float32)

        def zero_body(j, _):
            ds = pl.ds(j * LANES, LANES)
            o_cm[ds] = zf
            o_cmf[ds] = zf
            return 0

        lax.fori_loop(0, NCH, zero_body, 0, unroll=8)

        ones16 = jnp.ones((LANES,), jnp.float32)

        def p1(j, carry):
            c1, c2 = carry
            ds = pl.ds(j * LANES, LANES)
            flv = fl_v[ds]
            flfv = flf_v[ds]
            rank1 = plsc.cumsum(flv) - flv + c1
            rankf = plsc.cumsum(flfv) - flfv + c2
            rank1_v[ds] = rank1
            plsc.store_scatter(compf_v, [rankf], rmf_v[ds], mask=flfv == 1)
            plsc.store_scatter(o_cm, [bq_v[ds]], ones16, mask=flv == 1)
            plsc.store_scatter(o_cmf, [bqf_v[ds]], ones16, mask=flfv == 1)
            return (c1 + jnp.sum(flv), c2 + jnp.sum(flfv))

        lax.fori_loop(0, NCH, p1, (jnp.int32(0), jnp.int32(0)), unroll=4)

        def p2(j, _):
            ds = pl.ds(j * LANES, LANES)
            r1 = jnp.minimum(rank1_v[ds], HW - 1)
            cfv = plsc.load_gather(compf_v, [r1])
            olp = rm_v[ds] >= cfv
            flv = fl_v[ds]
            pix = lax.iota(jnp.int32, LANES) + j * LANES
            ok = jnp.logical_and(flv == 1, pix != 0)
            o_rm[ds] = jnp.where(jnp.logical_and(ok, olp), 1.0, 0.0)
            o_rmf[ds] = jnp.where(
                jnp.logical_and(ok, jnp.logical_not(olp)), 1.0, 0.0)
            return 0

        lax.fori_loop(0, NCH, p2, 0, unroll=4)

        pltpu.sync_copy(o_rm, orm_hbm)
        pltpu.sync_copy(o_rmf, ormf_hbm)
        pltpu.sync_copy(o_cm, ocm_hbm)
        pltpu.sync_copy(o_cmf, ocmf_hbm)


def _stage2(rm, rmf, bq, bqf, fl, flf):
    f32 = jnp.float32
    i32 = jnp.int32
    mesh = plsc.VectorSubcoreMesh(core_axis_name="c", subcore_axis_name="s")
    run = functools.partial(
        pl.kernel, mesh=mesh,
        compiler_params=pltpu.CompilerParams(needs_layout_passes=False),
        out_type=[jax.ShapeDtypeStruct((HW,), f32)] * 4,
        scratch_types=[
            pltpu.VMEM((HW,), f32),
            pltpu.VMEM((HW,), f32),
            pltpu.VMEM((HW,), i32),
            pltpu.VMEM((HW,), i32),
            pltpu.VMEM((HW,), i32),
            pltpu.VMEM((HW,), i32),
            pltpu.VMEM((HW,), i32),
            pltpu.VMEM((HW,), f32),
            pltpu.VMEM((HW,), f32),
            pltpu.VMEM((HW,), f32),
            pltpu.VMEM((HW,), f32),
            pltpu.VMEM((HW,), f32),
        ],
    )(_stage2_body)
    return run(rm, rmf, bq, bqf, fl, flf)


def _paste_body(lw_ref, lwf_ref, rm_ref, rmf_ref, cm_ref, cmf_ref, out_ref):
    s = lax.dot_general(lw_ref[...], cm_ref[...], (((1,), (1,)), ((), ())),
                        preferred_element_type=jnp.float32)
    sf = lax.dot_general(lwf_ref[...], cmf_ref[...], (((1,), (1,)), ((), ())),
                         preferred_element_type=jnp.float32)
    out_ref[...] = s * rm_ref[...] + sf * rmf_ref[...]


def _stage3(lwT, lwfT, rm2d, rmf2d, cm2d, cmf2d):
    return pl.pallas_call(
        _paste_body,
        out_shape=jax.ShapeDtypeStruct((C2, HW), jnp.float32),
    )(lwT, lwfT, rm2d, rmf2d, cm2d, cmf2d)


def kernel(input, flip_feat, flag, flag_flip):
    xf = input[0].reshape(C, HW)
    fwT = xf[:C2]
    lwT = xf[C2:]
    lwfT = flip_feat[0].reshape(C2, HW)
    fl = flag[0].astype(jnp.int32)
    flf = flag_flip[0].astype(jnp.int32)

    s1, b1, s2, b2 = _stage0(lwT, lwfT, fl.reshape(1, HW), flf.reshape(1, HW))
    rm3, bq3, rmf3, bqf3 = _stage1(fwT, lwT, lwfT, s1, b1, s2, b2)
    rm = rm3.reshape(HW)
    bq = bq3.reshape(HW)
    rmf = rmf3.reshape(HW)
    bqf = bqf3.reshape(HW)

    orm, ormf, ocm, ocmf = _stage2(rm, rmf, bq, bqf, fl, flf)

    shiftT = _stage3(lwT, lwfT, orm.reshape(1, HW), ormf.reshape(1, HW),
                     ocm.reshape(1, HW), ocmf.reshape(1, HW))

    return jnp.concatenate([xf, shiftT], axis=0).reshape(1, C + C2, H, W)


# speedup vs baseline: 1.1687x; 1.1054x over previous
import functools

import jax
import jax.numpy as jnp
from jax import lax
from jax.experimental import pallas as pl
from jax.experimental.pallas import tpu as pltpu
from jax.experimental.pallas import tpu_sc as plsc

B, C, H, W = 1, 512, 64, 64
HW = H * W
C2 = C // 2
RT = 512
NT = HW // RT
NEG_INF = float("-inf")
LANES = 16
NCH = HW // LANES


NB = HW // 128
MASK_BIAS = -1e30


def _cos_argmax_body(fw_ref, lw_ref, lwf_ref, fl_ref, flf_ref,
                     rm_ref, bq_ref, rmf_ref, bqf_ref,
                     s1_ref, b1_ref, s2_ref, b2_ref):
    @pl.when(pl.program_id(0) == 0)
    def _init():
        def side(lw, flags, s_ref, b_ref):
            nl = jnp.sum(lw * lw, axis=0)
            s_ref[...] = (1.0 / jnp.sqrt(nl))[None, :]
            b_ref[...] = jnp.where(flags == 0, 0.0, MASK_BIAS)

        side(lw_ref[...], fl_ref[...], s1_ref, b1_ref)
        side(lwf_ref[...], flf_ref[...], s2_ref, b2_ref)

    fw_t = fw_ref[...]
    nf = jnp.sum(fw_t * fw_t, axis=0)
    rnf = 1.0 / jnp.sqrt(nf)
    lane = lax.broadcasted_iota(jnp.int32, (RT, 128), 1)

    def one_side(lw, scale, bias):
        num = lax.dot_general(fw_t, lw, (((0,), (0,)), ((), ())),
                              preferred_element_type=jnp.float32)
        m_acc = jnp.full((RT, 128), NEG_INF, jnp.float32)
        b_acc = jnp.zeros((RT, 128), jnp.int32)
        for b in range(NB):
            sl = slice(b * 128, (b + 1) * 128)
            x = num[:, sl] * scale[0:1, sl] + bias[0:1, sl]
            better = x > m_acc
            m_acc = jnp.where(better, x, m_acc)
            b_acc = jnp.where(better, b, b_acc)
        i_acc = b_acc * 128 + lane
        m_t = m_acc.T
        i_t = i_acc.T
        m = jnp.max(m_t, axis=0)
        idx = jnp.min(jnp.where(m_t == m[None, :], i_t, HW), axis=0)
        return m * rnf, idx

    m, idx = one_side(lw_ref[...], s1_ref[...], b1_ref[...])
    mf, idxf = one_side(lwf_ref[...], s2_ref[...], b2_ref[...])
    rm_ref[...] = m.reshape(1, 1, RT)
    bq_ref[...] = idx.reshape(1, 1, RT)
    rmf_ref[...] = mf.reshape(1, 1, RT)
    bqf_ref[...] = idxf.reshape(1, 1, RT)


def _stage1(xf, lwfT, fl2d, flf2d):
    f32 = jnp.float32
    outs = [jax.ShapeDtypeStruct((NT, 1, RT), f32),
            jax.ShapeDtypeStruct((NT, 1, RT), jnp.int32),
            jax.ShapeDtypeStruct((NT, 1, RT), f32),
            jax.ShapeDtypeStruct((NT, 1, RT), jnp.int32)]
    in_specs = [
        pl.BlockSpec((C2, RT), lambda i: (0, i)),
        pl.BlockSpec((C2, HW), lambda i: (1, 0)),
        pl.BlockSpec((C2, HW), lambda i: (0, 0)),
        pl.BlockSpec((1, HW), lambda i: (0, 0)),
        pl.BlockSpec((1, HW), lambda i: (0, 0)),
    ]
    out_specs = [pl.BlockSpec((1, 1, RT), lambda i: (i, 0, 0))] * 4
    return pl.pallas_call(
        _cos_argmax_body, grid=(NT,), in_specs=in_specs,
        out_specs=out_specs, out_shape=outs,
        scratch_shapes=[
            pltpu.VMEM((1, HW), f32),
            pltpu.VMEM((1, HW), f32),
            pltpu.VMEM((1, HW), f32),
            pltpu.VMEM((1, HW), f32),
        ],
    )(xf, xf, lwfT, fl2d, flf2d)


_SC_PARAMS = dict(
    compiler_params=pltpu.CompilerParams(needs_layout_passes=False),
)


def _is_w0():
    return jnp.logical_and(lax.axis_index("c") == 0, lax.axis_index("s") == 0)


def _ranks_body(fl_hbm, flf_hbm, rank1_hbm, rankf_hbm,
                fl_v, flf_v, rank1_v, rankf_v):
    @pl.when(_is_w0())
    def _():
        pltpu.sync_copy(fl_hbm, fl_v)
        pltpu.sync_copy(flf_hbm, flf_v)

        def p(j, carry):
            c1, c2 = carry
            ds = pl.ds(j * LANES, LANES)
            flv = fl_v[ds]
            flfv = flf_v[ds]
            cs1 = plsc.cumsum(flv)
            cs2 = plsc.cumsum(flfv)
            rank1_v[ds] = cs1 - flv + c1
            rankf_v[ds] = cs2 - flfv + c2
            return (c1 + jnp.sum(flv), c2 + jnp.sum(flfv))

        lax.fori_loop(0, NCH, p, (jnp.int32(0), jnp.int32(0)), unroll=4)
        pltpu.sync_copy(rank1_v, rank1_hbm)
        pltpu.sync_copy(rankf_v, rankf_hbm)


def _stage2a(fl, flf):
    i32 = jnp.int32
    mesh = plsc.VectorSubcoreMesh(core_axis_name="c", subcore_axis_name="s")
    run = functools.partial(
        pl.kernel, mesh=mesh,
        out_type=[jax.ShapeDtypeStruct((HW,), i32)] * 2,
        scratch_types=[pltpu.VMEM((HW,), i32)] * 4,
        **_SC_PARAMS,
    )(_ranks_body)
    return run(fl, flf)


def _masks_body(rm_hbm, rmf_hbm, bq_hbm, bqf_hbm, fl_hbm, flf_hbm,
                rank1_hbm, rankf_hbm,
                orm_hbm, ormf_hbm, ocm_hbm, ocmf_hbm,
                rm_v, rmf_v, bq_v, bqf_v, fl_v, flf_v,
                rank1_v, rankf_v, compf_v, o_rm, o_rmf, o_cm, o_cmf):
    @pl.when(_is_w0())
    def _():
        pltpu.sync_copy(rm_hbm, rm_v)
        pltpu.sync_copy(rmf_hbm, rmf_v)
        pltpu.sync_copy(bq_hbm, bq_v)
        pltpu.sync_copy(bqf_hbm, bqf_v)
        pltpu.sync_copy(fl_hbm, fl_v)
        pltpu.sync_copy(flf_hbm, flf_v)
        pltpu.sync_copy(rank1_hbm, rank1_v)
        pltpu.sync_copy(rankf_hbm, rankf_v)

        zf = jnp.zeros((LANES,), jnp.float32)

        def zero_body(j, _):
            ds = pl.ds(j * LANES, LANES)
            o_cm[ds] = zf
            o_cmf[ds] = zf
            return 0

        lax.fori_loop(0, NCH, zero_body, 0, unroll=8)

        ones16 = jnp.ones((LANES,), jnp.float32)

        def p1(j, _):
            ds = pl.ds(j * LANES, LANES)
            flv = fl_v[ds]
            flfv = flf_v[ds]
            plsc.store_scatter(compf_v, [rankf_v[ds]], rmf_v[ds],
                               mask=flfv == 1)
            plsc.store_scatter(o_cm, [bq_v[ds]], ones16, mask=flv == 1)
            plsc.store_scatter(o_cmf, [bqf_v[ds]], ones16, mask=flfv == 1)
            return 0

        lax.fori_loop(0, NCH, p1, 0, unroll=4)

        def p2(j, _):
            ds = pl.ds(j * LANES, LANES)
            cfv = plsc.load_gather(compf_v, [rank1_v[ds]])
            olp = rm_v[ds] >= cfv
            flv = fl_v[ds]
            pix = lax.iota(jnp.int32, LANES) + j * LANES
            ok = jnp.logical_and(flv == 1, pix != 0)
            o_rm[ds] = jnp.where(jnp.logical_and(ok, olp), 1.0, 0.0)
            o_rmf[ds] = jnp.where(
                jnp.logical_and(ok, jnp.logical_not(olp)), 1.0, 0.0)
            return 0

        lax.fori_loop(0, NCH, p2, 0, unroll=4)

        pltpu.sync_copy(o_rm, orm_hbm)
        pltpu.sync_copy(o_rmf, ormf_hbm)
        pltpu.sync_copy(o_cm, ocm_hbm)
        pltpu.sync_copy(o_cmf, ocmf_hbm)


def _stage2b(rm, rmf, bq, bqf, fl, flf, rank1, rankf):
    f32 = jnp.float32
    i32 = jnp.int32
    mesh = plsc.VectorSubcoreMesh(core_axis_name="c", subcore_axis_name="s")
    run = functools.partial(
        pl.kernel, mesh=mesh,
        out_type=[jax.ShapeDtypeStruct((HW,), f32)] * 4,
        scratch_types=[
            pltpu.VMEM((HW,), f32),
            pltpu.VMEM((HW,), f32),
            pltpu.VMEM((HW,), i32),
            pltpu.VMEM((HW,), i32),
            pltpu.VMEM((HW,), i32),
            pltpu.VMEM((HW,), i32),
            pltpu.VMEM((HW,), i32),
            pltpu.VMEM((HW,), i32),
            pltpu.VMEM((HW,), f32),
            pltpu.VMEM((HW,), f32),
            pltpu.VMEM((HW,), f32),
            pltpu.VMEM((HW,), f32),
            pltpu.VMEM((HW,), f32),
        ],
        **_SC_PARAMS,
    )(_masks_body)
    return run(rm, rmf, bq, bqf, fl, flf, rank1, rankf)


def _paste_body(lw_ref, lwf_ref, rm_ref, rmf_ref, cm_ref, cmf_ref, out_ref):
    s = lax.dot_general(lw_ref[...], cm_ref[...], (((1,), (1,)), ((), ())),
                        preferred_element_type=jnp.float32)
    sf = lax.dot_general(lwf_ref[...], cmf_ref[...], (((1,), (1,)), ((), ())),
                         preferred_element_type=jnp.float32)
    out_ref[...] = s * rm_ref[...] + sf * rmf_ref[...]


def _stage3(xf, lwfT, rm2d, rmf2d, cm2d, cmf2d):
    in_specs = [
        pl.BlockSpec((C2, HW), lambda i: (1, 0)),
        pl.BlockSpec((C2, HW), lambda i: (0, 0)),
        pl.BlockSpec((1, HW), lambda i: (0, 0)),
        pl.BlockSpec((1, HW), lambda i: (0, 0)),
        pl.BlockSpec((1, HW), lambda i: (0, 0)),
        pl.BlockSpec((1, HW), lambda i: (0, 0)),
    ]
    return pl.pallas_call(
        _paste_body, grid=(1,), in_specs=in_specs,
        out_specs=pl.BlockSpec((C2, HW), lambda i: (0, 0)),
        out_shape=jax.ShapeDtypeStruct((C2, HW), jnp.float32),
    )(xf, lwfT, rm2d, rmf2d, cm2d, cmf2d)


def kernel(input, flip_feat, flag, flag_flip):
    xf = input[0].reshape(C, HW)
    lwfT = flip_feat[0].reshape(C2, HW)
    fl = flag[0].astype(jnp.int32)
    flf = flag_flip[0].astype(jnp.int32)

    rank1, rankf = _stage2a(fl, flf)

    rm3, bq3, rmf3, bqf3 = _stage1(xf, lwfT,
                                   fl.reshape(1, HW), flf.reshape(1, HW))
    rm = rm3.reshape(HW)
    bq = bq3.reshape(HW)
    rmf = rmf3.reshape(HW)
    bqf = bqf3.reshape(HW)

    orm, ormf, ocm, ocmf = _stage2b(rm, rmf, bq, bqf, fl, flf, rank1, rankf)

    shiftT = _stage3(xf, lwfT, orm.reshape(1, HW), ormf.reshape(1, HW),
                     ocm.reshape(1, HW), ocmf.reshape(1, HW))

    return jnp.concatenate([xf, shiftT], axis=0).reshape(1, C + C2, H, W)
